# baseline (speedup 1.0000x reference)
import functools

import jax
import jax.numpy as jnp
from jax import lax
from jax.experimental import pallas as pl
from jax.experimental.pallas import tpu as pltpu
from jax.experimental.pallas import tpu_sc as plsc

N = 16384
EPS = 1e-12
L = 16
NUM_CORES = 2
NUM_SUBCORES = 16
NUM_WORKERS = NUM_CORES * NUM_SUBCORES
BLOCK = 8192
VPB = BLOCK // L
UNROLL = 8


def _sc_partial_spmv(preds, vals, rows, cols, nnz, full):
    mesh = plsc.VectorSubcoreMesh(core_axis_name="c", subcore_axis_name="s")
    tstart = ((nnz - BLOCK) // 8) * 8
    tlen = nnz - tstart
    tvregs = -(-tlen // L)
    tbuf_len = tvregs * L
    tdelta = full * BLOCK - tstart

    @functools.partial(
        pl.kernel,
        out_type=jax.ShapeDtypeStruct((NUM_WORKERS, N), jnp.float32),
        mesh=mesh,
        compiler_params=pltpu.CompilerParams(needs_layout_passes=False,
                                             disable_bounds_checks=True),
        scratch_types=[
            pltpu.VMEM((N,), jnp.float32),
            pltpu.VMEM((N,), jnp.float32),
            pltpu.VMEM((BLOCK,), jnp.float32),
            pltpu.VMEM((BLOCK,), jnp.int32),
            pltpu.VMEM((BLOCK,), jnp.int32),
            pltpu.VMEM((BLOCK,), jnp.float32),
            pltpu.VMEM((BLOCK,), jnp.int32),
            pltpu.VMEM((BLOCK,), jnp.int32),
            pltpu.VMEM((tbuf_len,), jnp.float32),
            pltpu.VMEM((tbuf_len,), jnp.int32),
            pltpu.VMEM((tbuf_len,), jnp.int32),
            pltpu.SemaphoreType.DMA,
            pltpu.SemaphoreType.DMA,
            pltpu.SemaphoreType.DMA,
            pltpu.SemaphoreType.DMA,
        ],
    )
    def k(preds_hbm, vals_hbm, rows_hbm, cols_hbm, out_hbm,
          preds_v, ax_v, vals0, rows0, cols0, vals1, rows1, cols1,
          valst, rowst, colst, sem0, sem1, semt, psem):
        wid = lax.axis_index("s") * NUM_CORES + lax.axis_index("c")
        bufs = ((vals0, rows0, cols0, sem0), (vals1, rows1, cols1, sem1))
        is_tail_worker = wid == NUM_WORKERS - 1

        def start_block(bi, buf):
            vb, rb, cb, sem = buf
            base = jnp.minimum(bi, full - 1) * BLOCK
            pltpu.async_copy(vals_hbm.at[pl.ds(base, BLOCK)], vb, sem)
            pltpu.async_copy(rows_hbm.at[pl.ds(base, BLOCK)], rb, sem)
            pltpu.async_copy(cols_hbm.at[pl.ds(base, BLOCK)], cb, sem)

        def drain_block(buf):
            vb, rb, cb, sem = buf
            pltpu.make_async_copy(vals_hbm.at[pl.ds(0, BLOCK)], vb, sem).wait()
            pltpu.make_async_copy(rows_hbm.at[pl.ds(0, BLOCK)], rb, sem).wait()
            pltpu.make_async_copy(cols_hbm.at[pl.ds(0, BLOCK)], cb, sem).wait()

        lane = lax.iota(jnp.int32, L)
        shift_idx = jnp.minimum(lane + 1, L - 1)
        is_last = lane == (L - 1)
        not_last = lane < (L - 1)
        gdn = lax.GatherDimensionNumbers(
            offset_dims=(), collapsed_slice_dims=(0,), start_index_map=(0,))

        def process(buf):
            vb, rb, cb, _ = buf

            @plsc.parallel_loop(0, VPB, 1, unroll=UNROLL)
            def _(j):
                off = j * L
                v = vb[pl.ds(off, L)]
                r = rb[pl.ds(off, L)]
                c = cb[pl.ds(off, L)]
                p = plsc.load_gather(preds_v, [c])
                cs = plsc.cumsum(v * p)
                r_next = lax.gather(
                    r, shift_idx[:, None], gdn, slice_sizes=(1,),
                    mode=lax.GatherScatterMode.PROMISE_IN_BOUNDS)
                end = (r != r_next) | is_last
                plsc.addupdate_scatter(ax_v, [r], cs, mask=end)
                plsc.addupdate_scatter(ax_v, [r_next], -cs,
                                       mask=end & not_last)

        def process_tail():
            @plsc.parallel_loop(0, tvregs, 1, unroll=4)
            def _(j):
                off = j * L
                gidx = off + lane
                valid = (gidx >= tdelta) & (gidx < tlen)
                v = valst[pl.ds(off, L)]
                r = rowst[pl.ds(off, L)]
                c = colst[pl.ds(off, L)]
                cmask = jnp.where(valid, c, jnp.zeros((L,), jnp.int32))
                p = plsc.load_gather(preds_v, [cmask])
                cs = plsc.cumsum(
                    jnp.where(valid, v * p, jnp.zeros((L,), jnp.float32)))
                r_next = lax.gather(
                    r, shift_idx[:, None], gdn, slice_sizes=(1,),
                    mode=lax.GatherScatterMode.PROMISE_IN_BOUNDS)
                end = (r != r_next) | is_last | (gidx + 1 == tlen)
                plsc.addupdate_scatter(ax_v, [r], cs, mask=end & valid)
                plsc.addupdate_scatter(
                    ax_v, [r_next], -cs,
                    mask=end & not_last & valid & (gidx + 1 < tlen))

        bi0 = wid * full // NUM_WORKERS
        nb_w = (wid + 1) * full // NUM_WORKERS - bi0
        start_block(bi0, bufs[0])

        @pl.when(is_tail_worker)
        def _():
            pltpu.async_copy(vals_hbm.at[pl.ds(tstart, tlen)],
                             valst.at[pl.ds(0, tlen)], semt)
            pltpu.async_copy(rows_hbm.at[pl.ds(tstart, tlen)],
                             rowst.at[pl.ds(0, tlen)], semt)
            pltpu.async_copy(cols_hbm.at[pl.ds(tstart, tlen)],
                             colst.at[pl.ds(0, tlen)], semt)

        pcopy = pltpu.async_copy(preds_hbm, preds_v, psem)

        @plsc.parallel_loop(0, N // L, 1, unroll=UNROLL)
        def _(i):
            ax_v[pl.ds(i * L, L)] = jnp.zeros((L,), jnp.float32)

        pcopy.wait()

        def body(b, carry):
            nxt = bi0 + b + 1

            @pl.when(b % 2 == 0)
            def _():
                start_block(nxt, bufs[1])
                drain_block(bufs[0])
                process(bufs[0])

            @pl.when(b % 2 == 1)
            def _():
                start_block(nxt, bufs[0])
                drain_block(bufs[1])
                process(bufs[1])

            return carry

        lax.fori_loop(0, nb_w, body, 0)

        @pl.when(nb_w % 2 == 0)
        def _():
            drain_block(bufs[0])

        @pl.when(nb_w % 2 == 1)
        def _():
            drain_block(bufs[1])

        @pl.when(is_tail_worker)
        def _():
            pltpu.make_async_copy(vals_hbm.at[pl.ds(0, tlen)],
                                  valst.at[pl.ds(0, tlen)], semt).wait()
            pltpu.make_async_copy(rows_hbm.at[pl.ds(0, tlen)],
                                  rowst.at[pl.ds(0, tlen)], semt).wait()
            pltpu.make_async_copy(cols_hbm.at[pl.ds(0, tlen)],
                                  colst.at[pl.ds(0, tlen)], semt).wait()
            process_tail()

        pltpu.sync_copy(ax_v, out_hbm.at[wid])

    return k(preds, vals, rows, cols)


def _finish(partials, target):
    def body(p_hbm, t_ref, o_ref, p_v, sem):
        pltpu.async_copy(p_hbm, p_v, sem).wait()
        ax = jnp.sum(p_v[...], axis=0)
        t = t_ref[...].reshape(N)
        res = t - ax
        ss_res = jnp.sum(res * res)
        ss_t = jnp.sum(t * t)
        val = jnp.sqrt(ss_res) / (jnp.sqrt(ss_t) + EPS)
        o_ref[...] = jnp.full((1, 1), val, jnp.float32)

    return pl.pallas_call(
        body,
        in_specs=[pl.BlockSpec(memory_space=pl.ANY),
                  pl.BlockSpec(memory_space=pltpu.VMEM)],
        out_specs=pl.BlockSpec(memory_space=pltpu.VMEM),
        out_shape=jax.ShapeDtypeStruct((1, 1), jnp.float32),
        scratch_shapes=[pltpu.VMEM((NUM_WORKERS, N), jnp.float32),
                        pltpu.SemaphoreType.DMA],
    )(partials, target)


def kernel(preds, target_b, matrix_vals, matrix_rows, matrix_cols, batch_map):
    nnz = matrix_vals.shape[0]
    full = nnz // BLOCK
    if nnz % BLOCK == 0:
        full -= 1
    partials = _sc_partial_spmv(preds, matrix_vals, matrix_rows, matrix_cols,
                                nnz, full)
    out = _finish(partials, target_b.reshape(128, 128))
    return out[0, 0]


# speedup vs baseline: 1.0408x; 1.0408x over previous
import functools

import jax
import jax.numpy as jnp
from jax import lax
from jax.experimental import pallas as pl
from jax.experimental.pallas import tpu as pltpu
from jax.experimental.pallas import tpu_sc as plsc

N = 16384
EPS = 1e-12
L = 16
NUM_CORES = 2
NUM_SUBCORES = 16
NUM_WORKERS = NUM_CORES * NUM_SUBCORES
BLOCK = 8192
VPB = BLOCK // L
UNROLL = 8


def _sc_partial_spmv(preds, vals, rows, cols, nnz, full):
    mesh = plsc.VectorSubcoreMesh(core_axis_name="c", subcore_axis_name="s")
    tstart = ((nnz - BLOCK) // 8) * 8
    tlen = nnz - tstart
    tvregs = -(-tlen // L)
    tbuf_len = tvregs * L
    tdelta = full * BLOCK - tstart

    @functools.partial(
        pl.kernel,
        out_type=jax.ShapeDtypeStruct((NUM_WORKERS, N), jnp.float32),
        mesh=mesh,
        compiler_params=pltpu.CompilerParams(needs_layout_passes=False,
                                             disable_bounds_checks=True),
        scratch_types=[
            pltpu.VMEM((N,), jnp.float32),
            pltpu.VMEM((N,), jnp.float32),
            pltpu.VMEM((BLOCK,), jnp.float32),
            pltpu.VMEM((BLOCK,), jnp.int32),
            pltpu.VMEM((BLOCK,), jnp.int32),
            pltpu.VMEM((BLOCK,), jnp.float32),
            pltpu.VMEM((BLOCK,), jnp.int32),
            pltpu.VMEM((BLOCK,), jnp.int32),
            pltpu.VMEM((tbuf_len,), jnp.float32),
            pltpu.VMEM((tbuf_len,), jnp.int32),
            pltpu.VMEM((tbuf_len,), jnp.int32),
            pltpu.SemaphoreType.DMA,
            pltpu.SemaphoreType.DMA,
            pltpu.SemaphoreType.DMA,
            pltpu.SemaphoreType.DMA,
        ],
    )
    def k(preds_hbm, vals_hbm, rows_hbm, cols_hbm, out_hbm,
          preds_v, ax_v, vals0, rows0, cols0, vals1, rows1, cols1,
          valst, rowst, colst, sem0, sem1, semt, psem):
        wid = lax.axis_index("s") * NUM_CORES + lax.axis_index("c")
        bufs = ((vals0, rows0, cols0, sem0), (vals1, rows1, cols1, sem1))
        is_tail_worker = wid == NUM_WORKERS - 1

        def start_block(bi, buf):
            vb, rb, cb, sem = buf
            base = jnp.minimum(bi, full - 1) * BLOCK
            pltpu.async_copy(vals_hbm.at[pl.ds(base, BLOCK)], vb, sem)
            pltpu.async_copy(rows_hbm.at[pl.ds(base, BLOCK)], rb, sem)
            pltpu.async_copy(cols_hbm.at[pl.ds(base, BLOCK)], cb, sem)

        def drain_block(buf):
            vb, rb, cb, sem = buf
            pltpu.make_async_copy(vals_hbm.at[pl.ds(0, BLOCK)], vb, sem).wait()
            pltpu.make_async_copy(rows_hbm.at[pl.ds(0, BLOCK)], rb, sem).wait()
            pltpu.make_async_copy(cols_hbm.at[pl.ds(0, BLOCK)], cb, sem).wait()

        lane = lax.iota(jnp.int32, L)
        shift_idx = jnp.minimum(lane + 1, L - 1)
        is_last = lane == (L - 1)
        not_last = lane < (L - 1)
        gdn = lax.GatherDimensionNumbers(
            offset_dims=(), collapsed_slice_dims=(0,), start_index_map=(0,))

        def process(buf):
            vb, rb, cb, _ = buf

            @plsc.parallel_loop(0, VPB, 1, unroll=UNROLL)
            def _(j):
                off = j * L
                v = vb[pl.ds(off, L)]
                r = rb[pl.ds(off, L)]
                c = cb[pl.ds(off, L)]
                p = plsc.load_gather(preds_v, [c])
                cs = plsc.cumsum(v * p)
                r_next = lax.gather(
                    r, shift_idx[:, None], gdn, slice_sizes=(1,),
                    mode=lax.GatherScatterMode.PROMISE_IN_BOUNDS)
                end = (r != r_next) | is_last
                plsc.addupdate_scatter(ax_v, [r], cs, mask=end)
                plsc.addupdate_scatter(ax_v, [r_next], -cs,
                                       mask=end & not_last)

        def process_tail():
            @plsc.parallel_loop(0, tvregs, 1, unroll=4)
            def _(j):
                off = j * L
                gidx = off + lane
                valid = (gidx >= tdelta) & (gidx < tlen)
                v = valst[pl.ds(off, L)]
                r = rowst[pl.ds(off, L)]
                c = colst[pl.ds(off, L)]
                cmask = jnp.where(valid, c, jnp.zeros((L,), jnp.int32))
                p = plsc.load_gather(preds_v, [cmask])
                cs = plsc.cumsum(
                    jnp.where(valid, v * p, jnp.zeros((L,), jnp.float32)))
                r_next = lax.gather(
                    r, shift_idx[:, None], gdn, slice_sizes=(1,),
                    mode=lax.GatherScatterMode.PROMISE_IN_BOUNDS)
                end = (r != r_next) | is_last | (gidx + 1 == tlen)
                plsc.addupdate_scatter(ax_v, [r], cs, mask=end & valid)
                plsc.addupdate_scatter(
                    ax_v, [r_next], -cs,
                    mask=end & not_last & valid & (gidx + 1 < tlen))

        nblocks = full + 1
        bi0 = wid * nblocks // NUM_WORKERS
        ub = (wid + 1) * nblocks // NUM_WORKERS
        nb_w = jnp.minimum(ub, full) - bi0
        start_block(bi0, bufs[0])

        @pl.when(is_tail_worker)
        def _():
            pltpu.async_copy(vals_hbm.at[pl.ds(tstart, tlen)],
                             valst.at[pl.ds(0, tlen)], semt)
            pltpu.async_copy(rows_hbm.at[pl.ds(tstart, tlen)],
                             rowst.at[pl.ds(0, tlen)], semt)
            pltpu.async_copy(cols_hbm.at[pl.ds(tstart, tlen)],
                             colst.at[pl.ds(0, tlen)], semt)

        pcopy = pltpu.async_copy(preds_hbm, preds_v, psem)

        @plsc.parallel_loop(0, N // L, 1, unroll=UNROLL)
        def _(i):
            ax_v[pl.ds(i * L, L)] = jnp.zeros((L,), jnp.float32)

        pcopy.wait()

        def body(b, carry):
            nxt = bi0 + b + 1

            @pl.when(b % 2 == 0)
            def _():
                start_block(nxt, bufs[1])
                drain_block(bufs[0])
                process(bufs[0])

            @pl.when(b % 2 == 1)
            def _():
                start_block(nxt, bufs[0])
                drain_block(bufs[1])
                process(bufs[1])

            return carry

        lax.fori_loop(0, nb_w, body, 0)

        @pl.when(nb_w % 2 == 0)
        def _():
            drain_block(bufs[0])

        @pl.when(nb_w % 2 == 1)
        def _():
            drain_block(bufs[1])

        @pl.when(is_tail_worker)
        def _():
            pltpu.make_async_copy(vals_hbm.at[pl.ds(0, tlen)],
                                  valst.at[pl.ds(0, tlen)], semt).wait()
            pltpu.make_async_copy(rows_hbm.at[pl.ds(0, tlen)],
                                  rowst.at[pl.ds(0, tlen)], semt).wait()
            pltpu.make_async_copy(cols_hbm.at[pl.ds(0, tlen)],
                                  colst.at[pl.ds(0, tlen)], semt).wait()
            process_tail()

        pltpu.sync_copy(ax_v, out_hbm.at[wid])

    return k(preds, vals, rows, cols)


def _finish(partials, target):
    def body(p_hbm, t_ref, o_ref, p_v, sem):
        pltpu.async_copy(p_hbm, p_v, sem).wait()
        ax = jnp.sum(p_v[...], axis=0)
        t = t_ref[...].reshape(N)
        res = t - ax
        ss_res = jnp.sum(res * res)
        ss_t = jnp.sum(t * t)
        val = jnp.sqrt(ss_res) / (jnp.sqrt(ss_t) + EPS)
        o_ref[...] = jnp.full((1, 1), val, jnp.float32)

    return pl.pallas_call(
        body,
        in_specs=[pl.BlockSpec(memory_space=pl.ANY),
                  pl.BlockSpec(memory_space=pltpu.VMEM)],
        out_specs=pl.BlockSpec(memory_space=pltpu.VMEM),
        out_shape=jax.ShapeDtypeStruct((1, 1), jnp.float32),
        scratch_shapes=[pltpu.VMEM((NUM_WORKERS, N), jnp.float32),
                        pltpu.SemaphoreType.DMA],
    )(partials, target)


def kernel(preds, target_b, matrix_vals, matrix_rows, matrix_cols, batch_map):
    nnz = matrix_vals.shape[0]
    full = nnz // BLOCK
    if nnz % BLOCK == 0:
        full -= 1
    partials = _sc_partial_spmv(preds, matrix_vals, matrix_rows, matrix_cols,
                                nnz, full)
    out = _finish(partials, target_b.reshape(128, 128))
    return out[0, 0]
